# Initial kernel scaffold; baseline (speedup 1.0000x reference)
import jax
import jax.numpy as jnp
from jax.experimental import pallas as pl


def kernel(x, edge_index, edge_index2, W0l, b0l, W0r, W1l, b1l, W1r, Wlin, blin):
    raise NotImplementedError("write your pallas kernel here")


# speedup vs baseline: 3.0324x; 3.0324x over previous
import jax
import jax.numpy as jnp
from jax import lax
from jax.experimental import pallas as pl
from jax.experimental.pallas import tpu as pltpu
from jax.experimental.pallas import tpu_sc as plsc

N = 10000
D = 128
NCORE = 2
NSUB = 16
NW = NCORE * NSUB
K = 128
SB = 8
NB = 10
CH = SB * NB
EW = K * CH
EP = NW * EW
NP = 10112
NZB = NP // K
RPT = NP // NSUB


def _sc_agg_body(tab, srcs, dsts, out_acc, src_l, dst_l, rows_v, acc_sh, sem_g):
    c = lax.axis_index("c")
    s = lax.axis_index("s")
    wid = c * NSUB + s
    base = s * RPT

    def fill(i, _):
        zero16 = jnp.zeros((16,), jnp.float32)
        for jj in range(D // 16):
            rows_v[i, pl.ds(jj * 16, 16)] = zero16
        return 0
    lax.fori_loop(0, K, fill, 0)

    for r in range(5):
        b = r * NSUB + s

        @pl.when(b < NZB)
        def _():
            pltpu.sync_copy(rows_v, acc_sh.at[pl.ds(b * K, K)])

    plsc.subcore_barrier()

    def batch(bi, _):
        pltpu.sync_copy(srcs.at[wid, pl.ds(bi * SB, SB)], src_l)
        pltpu.sync_copy(dsts.at[wid, pl.ds(bi * SB, SB)], dst_l)
        for j in range(SB):
            pltpu.async_copy(tab.at[src_l.at[j]], rows_v, sem_g).wait()
            pltpu.sync_copy(rows_v, acc_sh.at[dst_l.at[j]], add=True)
        return 0
    lax.fori_loop(0, NB, batch, 0)

    plsc.subcore_barrier()

    pltpu.sync_copy(acc_sh.at[pl.ds(base, RPT)],
                    out_acc.at[c, pl.ds(base, RPT)])


def _sc_aggregate(tab, srcs, dsts):
    mesh = plsc.VectorSubcoreMesh(core_axis_name="c", subcore_axis_name="s")
    return pl.kernel(
        _sc_agg_body,
        out_type=jax.ShapeDtypeStruct((NCORE, NP, D), jnp.float32),
        mesh=mesh,
        scratch_types=[
            pltpu.VMEM((SB, K), jnp.int32),
            pltpu.VMEM((SB, K), jnp.int32),
            pltpu.VMEM((K, D), jnp.float32),
            pltpu.VMEM_SHARED((NP, D), jnp.float32),
            pltpu.SemaphoreType.DMA,
        ],
    )(tab, srcs, dsts)


NP2 = 2 * NP
CR = 256


def _sc_count_body(cdsts, out_cnt, dst_l, hist, idx_io, cnt_sh):
    c = lax.axis_index("c")
    s = lax.axis_index("s")
    wid = c * NSUB + s

    def fillz(i, _):
        zero16 = jnp.zeros((16,), jnp.float32)
        for jj in range(D // 16):
            hist[i, pl.ds(jj * 16, 16)] = zero16
        return 0
    lax.fori_loop(0, CR, fillz, 0)
    for g in range(2):
        for ks in range(8):
            idx_io[g, pl.ds(ks * 16, 16)] = (
                lax.iota(jnp.int32, 16) + (g * 128 + ks * 16))

    @pl.when(s == 0)
    def _():
        pltpu.sync_copy(hist, cnt_sh)

    plsc.subcore_barrier()

    ones16 = jnp.ones((16,), jnp.float32)

    def batch(bi, _):
        pltpu.sync_copy(cdsts.at[wid, pl.ds(bi * SB, SB)], dst_l)
        for j in range(SB):
            for g in range(K // 16):
                d16 = dst_l[j, pl.ds(g * 16, 16)]
                row16 = lax.shift_right_logical(d16, 7)
                col16 = lax.bitwise_and(d16, 127)
                plsc.addupdate_scatter(hist, [row16, col16], ones16)
        return 0
    lax.fori_loop(0, 2 * NB, batch, 0)

    for g in range(2):
        pltpu.sync_copy(hist.at[pl.ds(g * 128, 128)],
                        cnt_sh.at[idx_io.at[g]], add=True)

    plsc.subcore_barrier()

    @pl.when(s == 0)
    def _():
        pltpu.sync_copy(cnt_sh.at[pl.ds(0, 160)], out_cnt.at[c])


def _sc_count(cdsts):
    mesh = plsc.VectorSubcoreMesh(core_axis_name="c", subcore_axis_name="s")
    return pl.kernel(
        _sc_count_body,
        out_type=jax.ShapeDtypeStruct((NCORE, 160, D), jnp.float32),
        mesh=mesh,
        compiler_params=pltpu.CompilerParams(needs_layout_passes=False),
        scratch_types=[
            pltpu.VMEM((SB, K), jnp.int32),
            pltpu.VMEM((CR, D), jnp.float32),
            pltpu.VMEM((2, K), jnp.int32),
            pltpu.VMEM_SHARED((CR, D), jnp.float32),
        ],
    )(cdsts)


BR = 1000


def _mean_agg(acc_ref, cnt_ref):
    agg = acc_ref[0] + acc_ref[1]
    cnt = cnt_ref[0] + cnt_ref[1]
    scale = 1.0 / jnp.maximum(cnt, 1.0)
    return agg * scale


def _tc_layer0_body(acc_ref, cnt_ref, x_ref, wl_ref, bl_ref, wr_ref, o_ref):
    agg = _mean_agg(acc_ref, cnt_ref)
    h = (jnp.dot(agg, wl_ref[...], preferred_element_type=jnp.float32)
         + jnp.dot(x_ref[...], wr_ref[...], preferred_element_type=jnp.float32)
         + bl_ref[...])
    o_ref[...] = jnp.maximum(h, 0.0)


def _tc_final_body(acc_ref, cnt_ref, h_ref, wl_ref, bl_ref, wr_ref,
                   wlin_ref, blin_ref, o_ref):
    agg = _mean_agg(acc_ref, cnt_ref)
    h = (jnp.dot(agg, wl_ref[...], preferred_element_type=jnp.float32)
         + jnp.dot(h_ref[...], wr_ref[...], preferred_element_type=jnp.float32)
         + bl_ref[...])
    t = jnp.tanh(h)
    logits = (jnp.dot(t, wlin_ref[...], preferred_element_type=jnp.float32)
              + blin_ref[...])
    m = jnp.max(logits, axis=1, keepdims=True)
    e = jnp.exp(logits - m)
    o_ref[...] = e / jnp.sum(e, axis=1, keepdims=True)


_ACC_SPEC = pl.BlockSpec((NCORE, BR, D), lambda i: (0, i, 0))
_CNT_SPEC = pl.BlockSpec((NCORE, BR, 1), lambda i: (0, i, 0))
_ROW_SPEC = pl.BlockSpec((BR, D), lambda i: (i, 0))
_W_SPEC = pl.BlockSpec((D, D), lambda i: (0, 0))
_B_SPEC = pl.BlockSpec((1, D), lambda i: (0, 0))


def _tc_layer0(acc, cnt, x, wl, bl, wr):
    return pl.pallas_call(
        _tc_layer0_body,
        grid=(N // BR,),
        in_specs=[_ACC_SPEC, _CNT_SPEC, _ROW_SPEC, _W_SPEC, _B_SPEC, _W_SPEC],
        out_specs=_ROW_SPEC,
        out_shape=jax.ShapeDtypeStruct((N, D), jnp.float32),
    )(acc, cnt, x, wl, bl.reshape(1, D), wr)


def _tc_final(acc, cnt, h, wl, bl, wr, wlin, blin):
    return pl.pallas_call(
        _tc_final_body,
        grid=(N // BR,),
        in_specs=[_ACC_SPEC, _CNT_SPEC, _ROW_SPEC, _W_SPEC, _B_SPEC, _W_SPEC,
                  _W_SPEC, _B_SPEC],
        out_specs=_ROW_SPEC,
        out_shape=jax.ShapeDtypeStruct((N, D), jnp.float32),
    )(acc, cnt, h, wl, bl.reshape(1, D), wr, wlin, blin.reshape(1, D))


def _pad_edges(edge_index):
    src = edge_index[0]
    dst = edge_index[1]
    pad = EP - src.shape[0]
    srcp = jnp.concatenate(
        [src, jnp.zeros((pad,), jnp.int32)]).reshape(NW, CH, K)
    dstp = jnp.concatenate(
        [dst, jnp.full((pad,), NP - 1, jnp.int32)]).reshape(NW, CH, K)
    return srcp, dstp


@jax.jit
def kernel(x, edge_index, edge_index2, W0l, b0l, W0r, W1l, b1l, W1r, Wlin, blin):
    src0, dst0 = _pad_edges(edge_index)
    src1, dst1 = _pad_edges(edge_index2)

    cdsts = jnp.concatenate([dst0, dst1 + NP], axis=1)
    cnt_all = _sc_count(cdsts)
    cnt_flat = cnt_all.reshape(NCORE, 160 * D)
    cnt0 = cnt_flat[:, :NP].reshape(NCORE, NP, 1)
    cnt1 = cnt_flat[:, NP:NP2].reshape(NCORE, NP, 1)

    dep = (cnt_all[0, 0, 0] * 0.0).astype(jnp.int32)
    acc0 = _sc_aggregate(x, src0 + dep, dst0)
    h0 = _tc_layer0(acc0, cnt0, x, W0l, b0l, W0r)

    acc1 = _sc_aggregate(h0, src1, dst1)
    return _tc_final(acc1, cnt1, h0, W1l, b1l, W1r, Wlin, blin)


# speedup vs baseline: 3.3027x; 1.0891x over previous
import jax
import jax.numpy as jnp
from jax import lax
from jax.experimental import pallas as pl
from jax.experimental.pallas import tpu as pltpu
from jax.experimental.pallas import tpu_sc as plsc

N = 10000
D = 128
NCORE = 2
NSUB = 16
NW = NCORE * NSUB
K = 128
SB = 8
NB = 10
CH = SB * NB
EW = K * CH
EP = NW * EW
NP = 10112
NZB = NP // K
RPT = NP // NSUB


def _sc_agg_body(tab, srcs, dsts, out_acc,
                 src_l, dst_l, rows0, rows1, acc_sh,
                 sem_st, sem_g0, sem_g1, sem_s0, sem_s1):
    c = lax.axis_index("c")
    s = lax.axis_index("s")
    wid = c * NSUB + s
    base = s * RPT

    def fill(i, _):
        zero16 = jnp.zeros((16,), jnp.float32)
        for jj in range(D // 16):
            rows0[i, pl.ds(jj * 16, 16)] = zero16
        return 0
    lax.fori_loop(0, K, fill, 0)

    for r in range(5):
        b = r * NSUB + s

        @pl.when(b < NZB)
        def _():
            pltpu.sync_copy(rows0, acc_sh.at[pl.ds(b * K, K)])

    pltpu.sync_copy(srcs.at[wid, pl.ds(0, SB)], src_l.at[0])
    pltpu.sync_copy(dsts.at[wid, pl.ds(0, SB)], dst_l.at[0])
    pltpu.async_copy(tab.at[src_l.at[0, 0]], rows0, sem_g0)

    plsc.subcore_barrier()

    def outer(bi, _):
        p = bi % 2
        np_ = (bi + 1) % 2
        for jj in range(SB):
            j = bi * SB + jj
            if jj % 2 == 0:
                r_cur, g_cur, s_cur = rows0, sem_g0, sem_s0
                r_nxt, g_nxt, s_nxt = rows1, sem_g1, sem_s1
            else:
                r_cur, g_cur, s_cur = rows1, sem_g1, sem_s1
                r_nxt, g_nxt, s_nxt = rows0, sem_g0, sem_s0
            pltpu.make_async_copy(tab.at[src_l.at[p, jj]], r_cur, g_cur).wait()
            pltpu.async_copy(r_cur, acc_sh.at[dst_l.at[p, jj]], s_cur,
                             add=True)

            @pl.when(j >= 1)
            def _():
                pltpu.make_async_copy(
                    r_nxt, acc_sh.at[dst_l.at[p, jj]], s_nxt).wait()

            if jj == 0:
                @pl.when(bi + 1 < NB)
                def _():
                    pltpu.async_copy(
                        srcs.at[wid, pl.ds((bi + 1) * SB, SB)],
                        src_l.at[np_], sem_st)
                    pltpu.async_copy(
                        dsts.at[wid, pl.ds((bi + 1) * SB, SB)],
                        dst_l.at[np_], sem_st)
            if jj < SB - 1:
                pltpu.async_copy(tab.at[src_l.at[p, jj + 1]], r_nxt, g_nxt)
            else:
                @pl.when(bi + 1 < NB)
                def _():
                    pltpu.make_async_copy(
                        srcs.at[wid, pl.ds((bi + 1) * SB, SB)],
                        src_l.at[np_], sem_st).wait()
                    pltpu.make_async_copy(
                        dsts.at[wid, pl.ds((bi + 1) * SB, SB)],
                        dst_l.at[np_], sem_st).wait()
                    pltpu.async_copy(tab.at[src_l.at[np_, 0]], r_nxt, g_nxt)
        return 0
    lax.fori_loop(0, NB, outer, 0)

    pltpu.make_async_copy(rows1, acc_sh.at[dst_l.at[(NB - 1) % 2, SB - 1]],
                          sem_s1).wait()

    plsc.subcore_barrier()

    pltpu.sync_copy(acc_sh.at[pl.ds(base, RPT)],
                    out_acc.at[c, pl.ds(base, RPT)])


def _sc_aggregate(tab, srcs, dsts):
    mesh = plsc.VectorSubcoreMesh(core_axis_name="c", subcore_axis_name="s")
    return pl.kernel(
        _sc_agg_body,
        out_type=jax.ShapeDtypeStruct((NCORE, NP, D), jnp.float32),
        mesh=mesh,
        scratch_types=[
            pltpu.VMEM((2, SB, K), jnp.int32),
            pltpu.VMEM((2, SB, K), jnp.int32),
            pltpu.VMEM((K, D), jnp.float32),
            pltpu.VMEM((K, D), jnp.float32),
            pltpu.VMEM_SHARED((NP, D), jnp.float32),
            pltpu.SemaphoreType.DMA,
            pltpu.SemaphoreType.DMA,
            pltpu.SemaphoreType.DMA,
            pltpu.SemaphoreType.DMA,
            pltpu.SemaphoreType.DMA,
        ],
    )(tab, srcs, dsts)


NP2 = 2 * NP
CR = 256


def _sc_count_body(cdsts, out_cnt, dst_l, hist, idx_io, cnt_sh):
    c = lax.axis_index("c")
    s = lax.axis_index("s")
    wid = c * NSUB + s

    def fillz(i, _):
        zero16 = jnp.zeros((16,), jnp.float32)
        for jj in range(D // 16):
            hist[i, pl.ds(jj * 16, 16)] = zero16
        return 0
    lax.fori_loop(0, CR, fillz, 0)
    for g in range(2):
        for ks in range(8):
            idx_io[g, pl.ds(ks * 16, 16)] = (
                lax.iota(jnp.int32, 16) + (g * 128 + ks * 16))

    @pl.when(s == 0)
    def _():
        pltpu.sync_copy(hist, cnt_sh)

    plsc.subcore_barrier()

    ones16 = jnp.ones((16,), jnp.float32)

    def batch(bi, _):
        pltpu.sync_copy(cdsts.at[wid, pl.ds(bi * SB, SB)], dst_l)
        for j in range(SB):
            for g in range(K // 16):
                d16 = dst_l[j, pl.ds(g * 16, 16)]
                row16 = lax.shift_right_logical(d16, 7)
                col16 = lax.bitwise_and(d16, 127)
                plsc.addupdate_scatter(hist, [row16, col16], ones16)
        return 0
    lax.fori_loop(0, 2 * NB, batch, 0)

    for g in range(2):
        pltpu.sync_copy(hist.at[pl.ds(g * 128, 128)],
                        cnt_sh.at[idx_io.at[g]], add=True)

    plsc.subcore_barrier()

    @pl.when(s == 0)
    def _():
        pltpu.sync_copy(cnt_sh.at[pl.ds(0, 160)], out_cnt.at[c])


def _sc_count(cdsts):
    mesh = plsc.VectorSubcoreMesh(core_axis_name="c", subcore_axis_name="s")
    return pl.kernel(
        _sc_count_body,
        out_type=jax.ShapeDtypeStruct((NCORE, 160, D), jnp.float32),
        mesh=mesh,
        compiler_params=pltpu.CompilerParams(needs_layout_passes=False),
        scratch_types=[
            pltpu.VMEM((SB, K), jnp.int32),
            pltpu.VMEM((CR, D), jnp.float32),
            pltpu.VMEM((2, K), jnp.int32),
            pltpu.VMEM_SHARED((CR, D), jnp.float32),
        ],
    )(cdsts)


BR = 1000


def _mean_agg(acc_ref, cnt_ref):
    agg = acc_ref[0] + acc_ref[1]
    cnt = cnt_ref[0] + cnt_ref[1]
    scale = 1.0 / jnp.maximum(cnt, 1.0)
    return agg * scale


def _tc_layer0_body(acc_ref, cnt_ref, x_ref, wl_ref, bl_ref, wr_ref, o_ref):
    agg = _mean_agg(acc_ref, cnt_ref)
    h = (jnp.dot(agg, wl_ref[...], preferred_element_type=jnp.float32)
         + jnp.dot(x_ref[...], wr_ref[...], preferred_element_type=jnp.float32)
         + bl_ref[...])
    o_ref[...] = jnp.maximum(h, 0.0)


def _tc_final_body(acc_ref, cnt_ref, h_ref, wl_ref, bl_ref, wr_ref,
                   wlin_ref, blin_ref, o_ref):
    agg = _mean_agg(acc_ref, cnt_ref)
    h = (jnp.dot(agg, wl_ref[...], preferred_element_type=jnp.float32)
         + jnp.dot(h_ref[...], wr_ref[...], preferred_element_type=jnp.float32)
         + bl_ref[...])
    t = jnp.tanh(h)
    logits = (jnp.dot(t, wlin_ref[...], preferred_element_type=jnp.float32)
              + blin_ref[...])
    m = jnp.max(logits, axis=1, keepdims=True)
    e = jnp.exp(logits - m)
    o_ref[...] = e / jnp.sum(e, axis=1, keepdims=True)


_ACC_SPEC = pl.BlockSpec((NCORE, BR, D), lambda i: (0, i, 0))
_CNT_SPEC = pl.BlockSpec((NCORE, BR, 1), lambda i: (0, i, 0))
_ROW_SPEC = pl.BlockSpec((BR, D), lambda i: (i, 0))
_W_SPEC = pl.BlockSpec((D, D), lambda i: (0, 0))
_B_SPEC = pl.BlockSpec((1, D), lambda i: (0, 0))


def _tc_layer0(acc, cnt, x, wl, bl, wr):
    return pl.pallas_call(
        _tc_layer0_body,
        grid=(N // BR,),
        in_specs=[_ACC_SPEC, _CNT_SPEC, _ROW_SPEC, _W_SPEC, _B_SPEC, _W_SPEC],
        out_specs=_ROW_SPEC,
        out_shape=jax.ShapeDtypeStruct((N, D), jnp.float32),
    )(acc, cnt, x, wl, bl.reshape(1, D), wr)


def _tc_final(acc, cnt, h, wl, bl, wr, wlin, blin):
    return pl.pallas_call(
        _tc_final_body,
        grid=(N // BR,),
        in_specs=[_ACC_SPEC, _CNT_SPEC, _ROW_SPEC, _W_SPEC, _B_SPEC, _W_SPEC,
                  _W_SPEC, _B_SPEC],
        out_specs=_ROW_SPEC,
        out_shape=jax.ShapeDtypeStruct((N, D), jnp.float32),
    )(acc, cnt, h, wl, bl.reshape(1, D), wr, wlin, blin.reshape(1, D))


def _pad_edges(edge_index):
    src = edge_index[0]
    dst = edge_index[1]
    pad = EP - src.shape[0]
    srcp = jnp.concatenate(
        [src, jnp.zeros((pad,), jnp.int32)]).reshape(NW, CH, K)
    dstp = jnp.concatenate(
        [dst, jnp.full((pad,), NP - 1, jnp.int32)]).reshape(NW, CH, K)
    return srcp, dstp


@jax.jit
def kernel(x, edge_index, edge_index2, W0l, b0l, W0r, W1l, b1l, W1r, Wlin, blin):
    src0, dst0 = _pad_edges(edge_index)
    src1, dst1 = _pad_edges(edge_index2)

    cdsts = jnp.concatenate([dst0, dst1 + NP], axis=1)
    cnt_all = _sc_count(cdsts)
    cnt_flat = cnt_all.reshape(NCORE, 160 * D)
    cnt0 = cnt_flat[:, :NP].reshape(NCORE, NP, 1)
    cnt1 = cnt_flat[:, NP:NP2].reshape(NCORE, NP, 1)

    dep = (cnt_all[0, 0, 0] * 0.0).astype(jnp.int32)
    acc0 = _sc_aggregate(x, src0 + dep, dst0)
    h0 = _tc_layer0(acc0, cnt0, x, W0l, b0l, W0r)

    acc1 = _sc_aggregate(h0, src1, dst1)
    return _tc_final(acc1, cnt1, h0, W1l, b1l, W1r, Wlin, blin)


# speedup vs baseline: 3.4211x; 1.0359x over previous
import jax
import jax.numpy as jnp
from jax import lax
from jax.experimental import pallas as pl
from jax.experimental.pallas import tpu as pltpu
from jax.experimental.pallas import tpu_sc as plsc

N = 10000
D = 128
NCORE = 2
NSUB = 16
NW = NCORE * NSUB
K = 128
SB = 8
NB = 10
CH = SB * NB
EW = K * CH
EP = NW * EW
NP = 10112
NZB = NP // K
RPT = NP // NSUB


def _sc_agg_body(tab, srcs, dsts, out_acc,
                 src_l, dst_l, rows0, rows1, acc_sh,
                 sem_st, sem_g0, sem_g1, sem_s0, sem_s1):
    c = lax.axis_index("c")
    s = lax.axis_index("s")
    wid = c * NSUB + s
    base = s * RPT

    def fill(i, _):
        zero16 = jnp.zeros((16,), jnp.float32)
        for jj in range(D // 16):
            rows0[i, pl.ds(jj * 16, 16)] = zero16
        return 0
    lax.fori_loop(0, K, fill, 0)

    for r in range(5):
        b = r * NSUB + s

        @pl.when(b < NZB)
        def _():
            pltpu.sync_copy(rows0, acc_sh.at[pl.ds(b * K, K)])

    pltpu.sync_copy(srcs.at[wid, pl.ds(0, SB)], src_l.at[0])
    pltpu.sync_copy(dsts.at[wid, pl.ds(0, SB)], dst_l.at[0])
    pltpu.async_copy(tab.at[src_l.at[0, 0]], rows0, sem_g0)

    plsc.subcore_barrier()

    def outer(bi, _):
        p = bi % 2
        np_ = (bi + 1) % 2
        for jj in range(SB):
            j = bi * SB + jj
            if jj % 2 == 0:
                r_cur, g_cur, s_cur = rows0, sem_g0, sem_s0
                r_nxt, g_nxt, s_nxt = rows1, sem_g1, sem_s1
            else:
                r_cur, g_cur, s_cur = rows1, sem_g1, sem_s1
                r_nxt, g_nxt, s_nxt = rows0, sem_g0, sem_s0
            pltpu.make_async_copy(tab.at[src_l.at[p, jj]], r_cur, g_cur).wait()
            pltpu.async_copy(r_cur, acc_sh.at[dst_l.at[p, jj]], s_cur,
                             add=True)

            @pl.when(j >= 1)
            def _():
                pltpu.make_async_copy(
                    r_nxt, acc_sh.at[dst_l.at[p, jj]], s_nxt).wait()

            if jj == 0:
                @pl.when(bi + 1 < NB)
                def _():
                    pltpu.async_copy(
                        srcs.at[wid, pl.ds((bi + 1) * SB, SB)],
                        src_l.at[np_], sem_st)
                    pltpu.async_copy(
                        dsts.at[wid, pl.ds((bi + 1) * SB, SB)],
                        dst_l.at[np_], sem_st)
            if jj < SB - 1:
                pltpu.async_copy(tab.at[src_l.at[p, jj + 1]], r_nxt, g_nxt)
            else:
                @pl.when(bi + 1 < NB)
                def _():
                    pltpu.make_async_copy(
                        srcs.at[wid, pl.ds((bi + 1) * SB, SB)],
                        src_l.at[np_], sem_st).wait()
                    pltpu.make_async_copy(
                        dsts.at[wid, pl.ds((bi + 1) * SB, SB)],
                        dst_l.at[np_], sem_st).wait()
                    pltpu.async_copy(tab.at[src_l.at[np_, 0]], r_nxt, g_nxt)
        return 0
    lax.fori_loop(0, NB, outer, 0)

    pltpu.make_async_copy(rows1, acc_sh.at[dst_l.at[(NB - 1) % 2, SB - 1]],
                          sem_s1).wait()

    plsc.subcore_barrier()

    pltpu.sync_copy(acc_sh.at[pl.ds(base, RPT)],
                    out_acc.at[c, pl.ds(base, RPT)])


def _sc_aggregate(tab, srcs, dsts):
    mesh = plsc.VectorSubcoreMesh(core_axis_name="c", subcore_axis_name="s")
    return pl.kernel(
        _sc_agg_body,
        out_type=jax.ShapeDtypeStruct((NCORE, NP, D), jnp.float32),
        mesh=mesh,
        scratch_types=[
            pltpu.VMEM((2, SB, K), jnp.int32),
            pltpu.VMEM((2, SB, K), jnp.int32),
            pltpu.VMEM((K, D), jnp.float32),
            pltpu.VMEM((K, D), jnp.float32),
            pltpu.VMEM_SHARED((NP, D), jnp.float32),
            pltpu.SemaphoreType.DMA,
            pltpu.SemaphoreType.DMA,
            pltpu.SemaphoreType.DMA,
            pltpu.SemaphoreType.DMA,
            pltpu.SemaphoreType.DMA,
        ],
    )(tab, srcs, dsts)


NP2 = 2 * NP
CR = 256


def _sc_count_body(cdsts, out_cnt, dst_l, hist, idx_io, cnt_sh):
    c = lax.axis_index("c")
    s = lax.axis_index("s")
    wid = c * NSUB + s

    def fillz(i, _):
        zero16 = jnp.zeros((16,), jnp.float32)
        for jj in range(D // 16):
            hist[i, pl.ds(jj * 16, 16)] = zero16
        return 0
    lax.fori_loop(0, CR, fillz, 0)
    for g in range(2):
        for ks in range(8):
            idx_io[g, pl.ds(ks * 16, 16)] = (
                lax.iota(jnp.int32, 16) + (g * 128 + ks * 16))

    @pl.when(s == 0)
    def _():
        pltpu.sync_copy(hist, cnt_sh)

    plsc.subcore_barrier()

    ones16 = jnp.ones((16,), jnp.float32)

    def batch(bi, _):
        pltpu.sync_copy(cdsts.at[wid, pl.ds(bi * SB, SB)], dst_l)
        for j in range(SB):
            for g in range(K // 16):
                d16 = dst_l[j, pl.ds(g * 16, 16)]
                row16 = lax.shift_right_logical(d16, 7)
                col16 = lax.bitwise_and(d16, 127)
                plsc.addupdate_scatter(hist, [row16, col16], ones16)
        return 0
    lax.fori_loop(0, 2 * NB, batch, 0)

    for g in range(2):
        pltpu.sync_copy(hist.at[pl.ds(g * 128, 128)],
                        cnt_sh.at[idx_io.at[g]], add=True)

    plsc.subcore_barrier()

    @pl.when(s == 0)
    def _():
        pltpu.sync_copy(cnt_sh.at[pl.ds(0, 160)], out_cnt.at[c])


def _sc_count(cdsts):
    mesh = plsc.VectorSubcoreMesh(core_axis_name="c", subcore_axis_name="s")
    return pl.kernel(
        _sc_count_body,
        out_type=jax.ShapeDtypeStruct((NCORE, 160, D), jnp.float32),
        mesh=mesh,
        compiler_params=pltpu.CompilerParams(needs_layout_passes=False),
        scratch_types=[
            pltpu.VMEM((SB, K), jnp.int32),
            pltpu.VMEM((CR, D), jnp.float32),
            pltpu.VMEM((2, K), jnp.int32),
            pltpu.VMEM_SHARED((CR, D), jnp.float32),
        ],
    )(cdsts)


BR = 1000


def _mean_agg(acc_ref, cnt_ref):
    agg = acc_ref[0] + acc_ref[1]
    cnt = cnt_ref[0] + cnt_ref[1]
    scale = 1.0 / jnp.maximum(cnt, 1.0)
    return agg * scale


def _tc_layer0_body(acc_ref, cnt_ref, x_ref, wl_ref, bl_ref, wr_ref, o_ref):
    agg = _mean_agg(acc_ref, cnt_ref)
    h = (jnp.dot(agg, wl_ref[...], preferred_element_type=jnp.float32)
         + jnp.dot(x_ref[...], wr_ref[...], preferred_element_type=jnp.float32)
         + bl_ref[...])
    o_ref[...] = jnp.maximum(h, 0.0)


def _tc_final_body(acc_ref, cnt_ref, h_ref, wl_ref, bl_ref, wr_ref,
                   wlin_ref, blin_ref, o_ref):
    agg = _mean_agg(acc_ref, cnt_ref)
    h = (jnp.dot(agg, wl_ref[...], preferred_element_type=jnp.float32)
         + jnp.dot(h_ref[...], wr_ref[...], preferred_element_type=jnp.float32)
         + bl_ref[...])
    t = jnp.tanh(h)
    logits = (jnp.dot(t, wlin_ref[...], preferred_element_type=jnp.float32)
              + blin_ref[...])
    m = jnp.max(logits, axis=1, keepdims=True)
    e = jnp.exp(logits - m)
    o_ref[...] = e / jnp.sum(e, axis=1, keepdims=True)


_ACC_SPEC = pl.BlockSpec((NCORE, BR, D), lambda i: (0, i, 0))
_CNT_SPEC = pl.BlockSpec((NCORE, BR, 1), lambda i: (0, i, 0))
_ROW_SPEC = pl.BlockSpec((BR, D), lambda i: (i, 0))
_W_SPEC = pl.BlockSpec((D, D), lambda i: (0, 0))
_B_SPEC = pl.BlockSpec((1, D), lambda i: (0, 0))


def _tc_layer0(acc, cnt, x, wl, bl, wr):
    return pl.pallas_call(
        _tc_layer0_body,
        grid=(N // BR,),
        in_specs=[_ACC_SPEC, _CNT_SPEC, _ROW_SPEC, _W_SPEC, _B_SPEC, _W_SPEC],
        out_specs=_ROW_SPEC,
        out_shape=jax.ShapeDtypeStruct((N, D), jnp.float32),
    )(acc, cnt, x, wl, bl.reshape(1, D), wr)


def _tc_final(acc, cnt, h, wl, bl, wr, wlin, blin):
    return pl.pallas_call(
        _tc_final_body,
        grid=(N // BR,),
        in_specs=[_ACC_SPEC, _CNT_SPEC, _ROW_SPEC, _W_SPEC, _B_SPEC, _W_SPEC,
                  _W_SPEC, _B_SPEC],
        out_specs=_ROW_SPEC,
        out_shape=jax.ShapeDtypeStruct((N, D), jnp.float32),
    )(acc, cnt, h, wl, bl.reshape(1, D), wr, wlin, blin.reshape(1, D))


def _pad_edges(edge_index):
    src = edge_index[0]
    dst = edge_index[1]
    pad = EP - src.shape[0]
    srcp = jnp.concatenate(
        [src, jnp.zeros((pad,), jnp.int32)]).reshape(NW, CH, K)
    pad_dst = N + (jnp.arange(pad, dtype=jnp.int32) % (NP - N))
    dstp = jnp.concatenate([dst, pad_dst]).reshape(NW, CH, K)
    return srcp, dstp


@jax.jit
def kernel(x, edge_index, edge_index2, W0l, b0l, W0r, W1l, b1l, W1r, Wlin, blin):
    src0, dst0 = _pad_edges(edge_index)
    src1, dst1 = _pad_edges(edge_index2)

    cdsts = jnp.concatenate([dst0, dst1 + NP], axis=1)
    cnt_all = _sc_count(cdsts)
    cnt_flat = cnt_all.reshape(NCORE, 160 * D)
    cnt0 = cnt_flat[:, :NP].reshape(NCORE, NP, 1)
    cnt1 = cnt_flat[:, NP:NP2].reshape(NCORE, NP, 1)

    dep = (cnt_all[0, 0, 0] * 0.0).astype(jnp.int32)
    acc0 = _sc_aggregate(x, src0 + dep, dst0)
    h0 = _tc_layer0(acc0, cnt0, x, W0l, b0l, W0r)

    acc1 = _sc_aggregate(h0, src1, dst1)
    return _tc_final(acc1, cnt1, h0, W1l, b1l, W1r, Wlin, blin)
